# baseline (speedup 1.0000x reference)
import functools

import jax
import jax.numpy as jnp
import numpy as np
from jax import lax
from jax.experimental import pallas as pl
from jax.experimental.pallas import tpu as pltpu
from jax.experimental.pallas import tpu_sc as plsc

RATIO = 25
NUM_PATCHES_PER_AX = 14
NUM_PATCHES = NUM_PATCHES_PER_AX ** 2
NUM_CHANNELS = 10

T_TOTAL = NUM_PATCHES * NUM_CHANNELS
BATCH = 64
CHANS = 768
REMAIN_T = NUM_PATCHES * RATIO // 100

NUM_CORES = 2
NUM_SUBCORES = 16
NW = NUM_CORES * NUM_SUBCORES
ROWS = REMAIN_T * BATCH
ROWS_PER_W = 104


def _shuffle_indices(rng):
    idx_to_take = np.arange(0, NUM_PATCHES * NUM_CHANNELS, NUM_CHANNELS) + rng.choice(
        [0, 1, 2, 6], NUM_PATCHES)
    rest = np.delete(np.arange(NUM_PATCHES * NUM_CHANNELS), idx_to_take)
    rng.shuffle(rest)
    fwd = np.concatenate([idx_to_take, rest])
    bwd = np.argsort(fwd)
    return fwd, bwd


@functools.lru_cache(maxsize=1)
def _constant_indices():
    rng = np.random.default_rng(0)
    idxs = [_shuffle_indices(rng) for _ in range(BATCH)]
    fwd = np.stack([i[0] for i in idxs], axis=-1).astype(np.int32)
    bwd = np.stack([i[1] for i in idxs], axis=-1).astype(np.int32)
    g = (fwd[:REMAIN_T] * BATCH + np.arange(BATCH, dtype=np.int32)[None, :]).reshape(-1)
    return fwd.reshape(-1), bwd.reshape(-1), g


_mesh = plsc.VectorSubcoreMesh(
    core_axis_name="c", subcore_axis_name="s",
    num_cores=NUM_CORES, num_subcores=NUM_SUBCORES)


IDX_TOTAL = T_TOTAL * BATCH
IDX_PER_W = IDX_TOTAL // NW


@functools.partial(
    pl.kernel,
    out_type=(
        jax.ShapeDtypeStruct((ROWS, CHANS), jnp.float32),
        jax.ShapeDtypeStruct((IDX_TOTAL,), jnp.int32),
        jax.ShapeDtypeStruct((IDX_TOTAL,), jnp.int32),
    ),
    mesh=_mesh,
    scratch_types=[
        pltpu.VMEM((ROWS_PER_W,), jnp.int32),
        pltpu.VMEM((ROWS_PER_W, CHANS), jnp.float32),
        pltpu.VMEM((IDX_PER_W,), jnp.int32),
        pltpu.VMEM((IDX_PER_W,), jnp.int32),
        pltpu.SemaphoreType.DMA,
    ],
)
def _sc_gather(table_hbm, idx_hbm, fwd_hbm, bwd_hbm,
               out_hbm, fwd_out, bwd_out,
               idx_v, rows_v, fwd_v, bwd_v, sem):
    wid = lax.axis_index("s") * NUM_CORES + lax.axis_index("c")
    base = lax.min(96 * wid + 8 * lax.min(wid, 8), ROWS - ROWS_PER_W)
    pltpu.sync_copy(idx_hbm.at[pl.ds(base, ROWS_PER_W)], idx_v)
    g = pltpu.async_copy(table_hbm.at[idx_v], rows_v, sem)
    cbase = wid * IDX_PER_W
    pltpu.sync_copy(fwd_hbm.at[pl.ds(cbase, IDX_PER_W)], fwd_v)
    pltpu.sync_copy(fwd_v, fwd_out.at[pl.ds(cbase, IDX_PER_W)])
    pltpu.sync_copy(bwd_hbm.at[pl.ds(cbase, IDX_PER_W)], bwd_v)
    pltpu.sync_copy(bwd_v, bwd_out.at[pl.ds(cbase, IDX_PER_W)])
    g.wait()
    pltpu.sync_copy(rows_v, out_hbm.at[pl.ds(base, ROWS_PER_W)])


def kernel(patches):
    fwd, bwd, g = _constant_indices()
    table = patches.reshape(T_TOTAL * BATCH, CHANS)
    out, fwd_o, bwd_o = _sc_gather(
        table, jnp.asarray(g), jnp.asarray(fwd), jnp.asarray(bwd))
    return (out.reshape(REMAIN_T, BATCH, CHANS),
            fwd_o.reshape(T_TOTAL, BATCH),
            bwd_o.reshape(T_TOTAL, BATCH))


# speedup vs baseline: 1.2034x; 1.2034x over previous
import functools

import jax
import jax.numpy as jnp
import numpy as np
from jax import lax
from jax.experimental import pallas as pl
from jax.experimental.pallas import tpu as pltpu
from jax.experimental.pallas import tpu_sc as plsc

RATIO = 25
NUM_PATCHES_PER_AX = 14
NUM_PATCHES = NUM_PATCHES_PER_AX ** 2
NUM_CHANNELS = 10

T_TOTAL = NUM_PATCHES * NUM_CHANNELS
BATCH = 64
CHANS = 768
REMAIN_T = NUM_PATCHES * RATIO // 100

NUM_CORES = 2
NUM_SUBCORES = 16
NW = NUM_CORES * NUM_SUBCORES
ROWS = REMAIN_T * BATCH
ROWS_PER_W = 104


def _shuffle_indices(rng):
    idx_to_take = np.arange(0, NUM_PATCHES * NUM_CHANNELS, NUM_CHANNELS) + rng.choice(
        [0, 1, 2, 6], NUM_PATCHES)
    rest = np.delete(np.arange(NUM_PATCHES * NUM_CHANNELS), idx_to_take)
    rng.shuffle(rest)
    fwd = np.concatenate([idx_to_take, rest])
    bwd = np.argsort(fwd)
    return fwd, bwd


@functools.lru_cache(maxsize=1)
def _constant_indices():
    rng = np.random.default_rng(0)
    idxs = [_shuffle_indices(rng) for _ in range(BATCH)]
    fwd = np.stack([i[0] for i in idxs], axis=-1).astype(np.int32)
    bwd = np.stack([i[1] for i in idxs], axis=-1).astype(np.int32)
    g = (fwd[:REMAIN_T] * BATCH + np.arange(BATCH, dtype=np.int32)[None, :]).reshape(-1)
    return fwd.reshape(-1), bwd.reshape(-1), g


_mesh = plsc.VectorSubcoreMesh(
    core_axis_name="c", subcore_axis_name="s",
    num_cores=NUM_CORES, num_subcores=NUM_SUBCORES)


@functools.partial(
    pl.kernel,
    out_type=jax.ShapeDtypeStruct((ROWS, CHANS), jnp.float32),
    mesh=_mesh,
    scratch_types=[
        pltpu.VMEM((ROWS_PER_W,), jnp.int32),
        pltpu.VMEM((ROWS_PER_W, CHANS), jnp.float32),
        pltpu.SemaphoreType.DMA,
    ],
)
def _sc_gather(table_hbm, idx_hbm, out_hbm, idx_v, rows_v, sem):
    wid = lax.axis_index("s") * NUM_CORES + lax.axis_index("c")
    base = lax.min(96 * wid + 8 * lax.min(wid, 8), ROWS - ROWS_PER_W)
    pltpu.sync_copy(idx_hbm.at[pl.ds(base, ROWS_PER_W)], idx_v)
    pltpu.async_copy(table_hbm.at[idx_v], rows_v, sem).wait()
    pltpu.sync_copy(rows_v, out_hbm.at[pl.ds(base, ROWS_PER_W)])


def kernel(patches):
    fwd, bwd, g = _constant_indices()
    table = patches.reshape(T_TOTAL * BATCH, CHANS)
    out = _sc_gather(table, jnp.asarray(g))
    return (out.reshape(REMAIN_T, BATCH, CHANS),
            jnp.asarray(fwd).reshape(T_TOTAL, BATCH),
            jnp.asarray(bwd).reshape(T_TOTAL, BATCH))


# speedup vs baseline: 1.2274x; 1.0199x over previous
import functools

import jax
import jax.numpy as jnp
import numpy as np
from jax import lax
from jax.experimental import pallas as pl
from jax.experimental.pallas import tpu as pltpu
from jax.experimental.pallas import tpu_sc as plsc

RATIO = 25
NUM_PATCHES_PER_AX = 14
NUM_PATCHES = NUM_PATCHES_PER_AX ** 2
NUM_CHANNELS = 10

T_TOTAL = NUM_PATCHES * NUM_CHANNELS
BATCH = 64
CHANS = 768
REMAIN_T = NUM_PATCHES * RATIO // 100

NUM_CORES = 2
NUM_SUBCORES = 16
NW = NUM_CORES * NUM_SUBCORES
ROWS = REMAIN_T * BATCH
ROWS_PER_W = 104


def _shuffle_indices(rng):
    idx_to_take = np.arange(0, NUM_PATCHES * NUM_CHANNELS, NUM_CHANNELS) + rng.choice(
        [0, 1, 2, 6], NUM_PATCHES)
    rest = np.delete(np.arange(NUM_PATCHES * NUM_CHANNELS), idx_to_take)
    rng.shuffle(rest)
    fwd = np.concatenate([idx_to_take, rest])
    bwd = np.argsort(fwd)
    return fwd, bwd


@functools.lru_cache(maxsize=1)
def _constant_indices():
    rng = np.random.default_rng(0)
    idxs = [_shuffle_indices(rng) for _ in range(BATCH)]
    fwd = np.stack([i[0] for i in idxs], axis=-1).astype(np.int32)
    bwd = np.stack([i[1] for i in idxs], axis=-1).astype(np.int32)
    g = (fwd[:REMAIN_T] * BATCH + np.arange(BATCH, dtype=np.int32)[None, :]).reshape(-1)
    return fwd.reshape(-1), bwd.reshape(-1), g


_mesh = plsc.VectorSubcoreMesh(
    core_axis_name="c", subcore_axis_name="s",
    num_cores=NUM_CORES, num_subcores=NUM_SUBCORES)


@functools.partial(
    pl.kernel,
    out_type=jax.ShapeDtypeStruct((ROWS, CHANS), jnp.float32),
    mesh=_mesh,
    scratch_types=[
        pltpu.VMEM((ROWS_PER_W,), jnp.int32),
        pltpu.VMEM((ROWS_PER_W, CHANS), jnp.float32),
        pltpu.SemaphoreType.DMA,
    ],
)
def _sc_gather(table_hbm, idx_hbm, out_hbm, idx_v, rows_v, sem):
    wid = lax.axis_index("s") * NUM_CORES + lax.axis_index("c")
    base = lax.min(96 * wid + 8 * lax.min(wid, 8), ROWS - ROWS_PER_W)
    pltpu.sync_copy(idx_hbm.at[pl.ds(base, ROWS_PER_W)], idx_v)
    pltpu.async_copy(table_hbm.at[idx_v], rows_v, sem).wait()
    pltpu.sync_copy(rows_v, out_hbm.at[pl.ds(base, ROWS_PER_W)])


def _tc_passthrough(fwd_ref, bwd_ref, fwd_out, bwd_out):
    fwd_out[...] = fwd_ref[...]
    bwd_out[...] = bwd_ref[...]


def _emit_index_outputs(fwd, bwd):
    return pl.pallas_call(
        _tc_passthrough,
        out_shape=(
            jax.ShapeDtypeStruct((T_TOTAL, BATCH), jnp.int32),
            jax.ShapeDtypeStruct((T_TOTAL, BATCH), jnp.int32),
        ),
    )(fwd, bwd)


def kernel(patches):
    fwd, bwd, g = _constant_indices()
    table = patches.reshape(T_TOTAL * BATCH, CHANS)
    out = _sc_gather(table, jnp.asarray(g))
    fwd_o, bwd_o = _emit_index_outputs(
        jnp.asarray(fwd).reshape(T_TOTAL, BATCH),
        jnp.asarray(bwd).reshape(T_TOTAL, BATCH))
    return (out.reshape(REMAIN_T, BATCH, CHANS), fwd_o, bwd_o)
